# baseline (speedup 1.0000x reference)
import functools

import jax
import jax.numpy as jnp
import numpy as np
from jax import lax
from jax.experimental import pallas as pl
from jax.experimental.pallas import tpu as pltpu
from jax.experimental.pallas import tpu_sc as plsc

B = 16384
D = 256
NC = 2
NS = 16
NW = NC * NS
RPW = B // NW
CH = 64
NCHUNK = RPW // CH
NB = 3
VC = 250
VS = 500
KREP = 4

_i500 = np.arange(VS)
_RO = (_i500[:, None] // 50 == np.arange(10)[None, :]).astype(np.float32)
_RB = ((_i500[:, None] // 5) % 10 == np.arange(10)[None, :]).astype(np.float32)
_RD = (_i500[:, None] % 5 == np.arange(5)[None, :]).astype(np.float32)


def _tables_body(ct_ref, ot_ref, bt_ref, dt_ref, w_ref, b_ref,
                 ro_ref, rb_ref, rd_ref, pc_ref, ps_ref):
    w = w_ref[...]
    dn = (((1,), (1,)), ((), ()))
    pc = lax.dot_general(ct_ref[...], w[:, 0:32], dn,
                         preferred_element_type=jnp.float32)
    for k in range(KREP):
        pc_ref[pl.ds(k * VC, VC), :] = pc
    po = lax.dot_general(ot_ref[...], w[:, 32:48], dn,
                         preferred_element_type=jnp.float32)
    pb = lax.dot_general(bt_ref[...], w[:, 48:64], dn,
                         preferred_element_type=jnp.float32)
    pd = lax.dot_general(dt_ref[...], w[:, 64:72], dn,
                         preferred_element_type=jnp.float32)
    dn2 = (((1,), (0,)), ((), ()))
    ps = (lax.dot_general(ro_ref[...], po, dn2, preferred_element_type=jnp.float32)
          + lax.dot_general(rb_ref[...], pb, dn2, preferred_element_type=jnp.float32)
          + lax.dot_general(rd_ref[...], pd, dn2, preferred_element_type=jnp.float32)
          + b_ref[...])
    for k in range(KREP):
        ps_ref[pl.ds(k * VS, VS), :] = ps


def _build_tables(ct, ot, bt, dt, w, b):
    return pl.pallas_call(
        _tables_body,
        out_shape=(
            jax.ShapeDtypeStruct((KREP * VC, D), jnp.float32),
            jax.ShapeDtypeStruct((KREP * VS, D), jnp.float32),
        ),
    )(ct, ot, bt, dt, w, b.reshape(1, D), jnp.asarray(_RO), jnp.asarray(_RB),
      jnp.asarray(_RD))


_SC_ENCODE_CACHE = []


def _get_sc_encode():
    if _SC_ENCODE_CACHE:
        return _SC_ENCODE_CACHE[0]
    mesh = plsc.VectorSubcoreMesh(core_axis_name="c", subcore_axis_name="s",
                                  num_cores=NC, num_subcores=NS)

    @functools.partial(
        pl.kernel,
        out_type=jax.ShapeDtypeStruct((B, D), jnp.float32),
        mesh=mesh,
        scratch_types=[
            pltpu.VMEM((RPW,), jnp.int32),
            pltpu.VMEM((RPW,), jnp.int32),
            pltpu.VMEM((RPW,), jnp.int32),
            pltpu.VMEM((RPW,), jnp.int32),
            pltpu.VMEM((RPW,), jnp.int32),
            pltpu.VMEM((NB, CH, D), jnp.float32),
            pltpu.VMEM((NB, CH, D), jnp.float32),
            pltpu.SemaphoreType.DMA,
            pltpu.SemaphoreType.DMA,
            pltpu.SemaphoreType.DMA,
            pltpu.SemaphoreType.DMA,
            pltpu.SemaphoreType.DMA,
            pltpu.SemaphoreType.DMA,
            pltpu.SemaphoreType.DMA,
            pltpu.SemaphoreType.DMA,
            pltpu.SemaphoreType.DMA,
            pltpu.SemaphoreType.DMA,
        ],
    )
    def _sc_encode(pc_hbm, ps_hbm, country_hbm, os_hbm, br_hbm, dv_hbm, out_hbm,
                   idc, ids, sto, stb, std, bufa, bufb,
                   sa0, sa1, sa2, sb0, sb1, sb2, so0, so1, so2, sst):
        wid = lax.axis_index("s") * NC + lax.axis_index("c")
        base = wid * RPW
        stg = [
            pltpu.async_copy(country_hbm.at[pl.ds(base, RPW)], idc, sst),
            pltpu.async_copy(os_hbm.at[pl.ds(base, RPW)], sto, sst),
            pltpu.async_copy(br_hbm.at[pl.ds(base, RPW)], stb, sst),
            pltpu.async_copy(dv_hbm.at[pl.ds(base, RPW)], std, sst),
        ]
        for h in stg:
            h.wait()
        coff = jnp.broadcast_to((wid % KREP) * VC, (16,)).astype(jnp.int32)
        soff = jnp.broadcast_to((wid % KREP) * VS, (16,)).astype(jnp.int32)
        for i in range(RPW // 16):
            sl = pl.ds(i * 16, 16)
            idc[sl] = idc[sl] + coff
            ids[sl] = sto[sl] * 50 + stb[sl] * 5 + std[sl] + soff

        sem_a, sem_b, sem_o = (sa0, sa1, sa2), (sb0, sb1, sb2), (so0, so1, so2)

        def fire(ch):
            s = ch % NB
            isl = pl.ds(ch * CH, CH)
            return (pltpu.async_copy(pc_hbm.at[idc.at[isl]], bufa.at[s], sem_a[s]),
                    pltpu.async_copy(ps_hbm.at[ids.at[isl]], bufb.at[s], sem_b[s]))

        gathers = {}
        stores = {}
        for ch in range(min(NB - 1, NCHUNK)):
            gathers[ch] = fire(ch)
        for ch in range(NCHUNK):
            s = ch % NB
            if ch + NB - 1 < NCHUNK:
                if ch >= 1:
                    stores[ch - 1].wait()
                gathers[ch + NB - 1] = fire(ch + NB - 1)
            ga, gb = gathers[ch]
            ga.wait()
            gb.wait()

            def _add_row(r, _, s=s):
                for j in range(D // 16):
                    sl2 = pl.ds(j * 16, 16)
                    plsc.addupdate(bufa.at[s, r, sl2], bufb[s, r, sl2])
                return 0

            lax.fori_loop(0, CH, _add_row, 0, unroll=2)
            stores[ch] = pltpu.async_copy(
                bufa.at[s], out_hbm.at[pl.ds(base + ch * CH, CH)], sem_o[s])
        for ch in range(max(0, NCHUNK - NB), NCHUNK):
            stores[ch].wait()

    _SC_ENCODE_CACHE.append(_sc_encode)
    return _sc_encode


def kernel(country, os, browser, device_type, country_table, os_table,
           browser_table, device_type_table, W, b):
    pc, ps = _build_tables(country_table, os_table, browser_table,
                           device_type_table, W, b)
    return _get_sc_encode()(
        pc, ps,
        country.astype(jnp.int32), os.astype(jnp.int32),
        browser.astype(jnp.int32), device_type.astype(jnp.int32))


# speedup vs baseline: 1.0181x; 1.0181x over previous
import functools

import jax
import jax.numpy as jnp
import numpy as np
from jax import lax
from jax.experimental import pallas as pl
from jax.experimental.pallas import tpu as pltpu
from jax.experimental.pallas import tpu_sc as plsc

B = 16384
D = 256
NC = 2
NS = 16
NW = NC * NS
RPW = B // NW
CH = 64
NCHUNK = RPW // CH
NB = 3
VC = 250
VS = 500
KREP = 8

_i500 = np.arange(VS)
_RO = (_i500[:, None] // 50 == np.arange(10)[None, :]).astype(np.float32)
_RB = ((_i500[:, None] // 5) % 10 == np.arange(10)[None, :]).astype(np.float32)
_RD = (_i500[:, None] % 5 == np.arange(5)[None, :]).astype(np.float32)


def _tables_body(ct_ref, ot_ref, bt_ref, dt_ref, w_ref, b_ref,
                 ro_ref, rb_ref, rd_ref, pc_ref, ps_ref):
    w = w_ref[...]
    dn = (((1,), (1,)), ((), ()))
    pc = lax.dot_general(ct_ref[...], w[:, 0:32], dn,
                         preferred_element_type=jnp.float32)
    for k in range(KREP):
        pc_ref[pl.ds(k * VC, VC), :] = pc
    po = lax.dot_general(ot_ref[...], w[:, 32:48], dn,
                         preferred_element_type=jnp.float32)
    pb = lax.dot_general(bt_ref[...], w[:, 48:64], dn,
                         preferred_element_type=jnp.float32)
    pd = lax.dot_general(dt_ref[...], w[:, 64:72], dn,
                         preferred_element_type=jnp.float32)
    dn2 = (((1,), (0,)), ((), ()))
    ps = (lax.dot_general(ro_ref[...], po, dn2, preferred_element_type=jnp.float32)
          + lax.dot_general(rb_ref[...], pb, dn2, preferred_element_type=jnp.float32)
          + lax.dot_general(rd_ref[...], pd, dn2, preferred_element_type=jnp.float32)
          + b_ref[...])
    for k in range(KREP):
        ps_ref[pl.ds(k * VS, VS), :] = ps


def _build_tables(ct, ot, bt, dt, w, b):
    return pl.pallas_call(
        _tables_body,
        out_shape=(
            jax.ShapeDtypeStruct((KREP * VC, D), jnp.float32),
            jax.ShapeDtypeStruct((KREP * VS, D), jnp.float32),
        ),
    )(ct, ot, bt, dt, w, b.reshape(1, D), jnp.asarray(_RO), jnp.asarray(_RB),
      jnp.asarray(_RD))


_SC_ENCODE_CACHE = []


def _get_sc_encode():
    if _SC_ENCODE_CACHE:
        return _SC_ENCODE_CACHE[0]
    mesh = plsc.VectorSubcoreMesh(core_axis_name="c", subcore_axis_name="s",
                                  num_cores=NC, num_subcores=NS)

    @functools.partial(
        pl.kernel,
        out_type=jax.ShapeDtypeStruct((B, D), jnp.float32),
        mesh=mesh,
        scratch_types=[
            pltpu.VMEM((RPW,), jnp.int32),
            pltpu.VMEM((RPW,), jnp.int32),
            pltpu.VMEM((RPW,), jnp.int32),
            pltpu.VMEM((RPW,), jnp.int32),
            pltpu.VMEM((RPW,), jnp.int32),
            pltpu.VMEM((NB, CH, D), jnp.float32),
            pltpu.VMEM((NB, CH, D), jnp.float32),
            pltpu.SemaphoreType.DMA,
            pltpu.SemaphoreType.DMA,
            pltpu.SemaphoreType.DMA,
            pltpu.SemaphoreType.DMA,
            pltpu.SemaphoreType.DMA,
            pltpu.SemaphoreType.DMA,
            pltpu.SemaphoreType.DMA,
            pltpu.SemaphoreType.DMA,
            pltpu.SemaphoreType.DMA,
            pltpu.SemaphoreType.DMA,
        ],
    )
    def _sc_encode(pc_hbm, ps_hbm, country_hbm, os_hbm, br_hbm, dv_hbm, out_hbm,
                   idc, ids, sto, stb, std, bufa, bufb,
                   sa0, sa1, sa2, sb0, sb1, sb2, so0, so1, so2, sst):
        wid = lax.axis_index("s") * NC + lax.axis_index("c")
        base = wid * RPW
        stg = [
            pltpu.async_copy(country_hbm.at[pl.ds(base, RPW)], idc, sst),
            pltpu.async_copy(os_hbm.at[pl.ds(base, RPW)], sto, sst),
            pltpu.async_copy(br_hbm.at[pl.ds(base, RPW)], stb, sst),
            pltpu.async_copy(dv_hbm.at[pl.ds(base, RPW)], std, sst),
        ]
        for h in stg:
            h.wait()
        coff = jnp.broadcast_to((wid % KREP) * VC, (16,)).astype(jnp.int32)
        soff = jnp.broadcast_to((wid % KREP) * VS, (16,)).astype(jnp.int32)
        for i in range(RPW // 16):
            sl = pl.ds(i * 16, 16)
            idc[sl] = idc[sl] + coff
            ids[sl] = sto[sl] * 50 + stb[sl] * 5 + std[sl] + soff

        sem_a, sem_b, sem_o = (sa0, sa1, sa2), (sb0, sb1, sb2), (so0, so1, so2)

        def fire(ch):
            s = ch % NB
            isl = pl.ds(ch * CH, CH)
            return (pltpu.async_copy(pc_hbm.at[idc.at[isl]], bufa.at[s], sem_a[s]),
                    pltpu.async_copy(ps_hbm.at[ids.at[isl]], bufb.at[s], sem_b[s]))

        gathers = {}
        stores = {}
        for ch in range(min(NB - 1, NCHUNK)):
            gathers[ch] = fire(ch)
        for ch in range(NCHUNK):
            s = ch % NB
            if ch + NB - 1 < NCHUNK:
                if ch >= 1:
                    stores[ch - 1].wait()
                gathers[ch + NB - 1] = fire(ch + NB - 1)
            ga, gb = gathers[ch]
            ga.wait()
            gb.wait()

            @plsc.parallel_loop(0, CH, 1, unroll=2)
            def _add_row(r, s=s):
                for j in range(D // 16):
                    sl2 = pl.ds(j * 16, 16)
                    plsc.addupdate(bufa.at[s, r, sl2], bufb[s, r, sl2])
            stores[ch] = pltpu.async_copy(
                bufa.at[s], out_hbm.at[pl.ds(base + ch * CH, CH)], sem_o[s])
        for ch in range(max(0, NCHUNK - NB), NCHUNK):
            stores[ch].wait()

    _SC_ENCODE_CACHE.append(_sc_encode)
    return _sc_encode


def kernel(country, os, browser, device_type, country_table, os_table,
           browser_table, device_type_table, W, b):
    pc, ps = _build_tables(country_table, os_table, browser_table,
                           device_type_table, W, b)
    return _get_sc_encode()(
        pc, ps,
        country.astype(jnp.int32), os.astype(jnp.int32),
        browser.astype(jnp.int32), device_type.astype(jnp.int32))
